# Initial kernel scaffold; baseline (speedup 1.0000x reference)
import jax
import jax.numpy as jnp
from jax.experimental import pallas as pl


def kernel(x, edge_index, batch, g1w1, g1b1, g1w2, g1b2, g2w1, g2b1, g2w2, g2b2, pw1, pb1, pw2, pb2, ow1, ob1, ow2, ob2):
    raise NotImplementedError("write your pallas kernel here")


# speedup vs baseline: 2.8626x; 2.8626x over previous
import functools

import jax
import jax.numpy as jnp
from jax import lax
from jax.experimental import pallas as pl
from jax.experimental.pallas import tpu as pltpu
from jax.experimental.pallas import tpu_sc as plsc

N = 10000
E = 320000
F = 128
H = 256
S = 32
NG = 8
L = 128

NC = 2
NS = 16
QW = 64
EPT = E // NS
C = 80
NCH = EPT // C
RPT = 624
ZR = 78
DW = 16


def _dg(x, w):
    return lax.dot_general(x, w, (((1,), (1,)), ((), ())),
                           preferred_element_type=jnp.float32)


def _dg_hi(x, w):
    return lax.dot_general(x, w, (((1,), (1,)), ((), ())),
                           precision=lax.Precision.HIGHEST,
                           preferred_element_type=jnp.float32)


def _make_sc_edge(with_deg: bool):
    mesh = plsc.VectorSubcoreMesh(core_axis_name="c", subcore_axis_name="s")
    out_type = [jax.ShapeDtypeStruct((NC * N, QW), jnp.float32)]
    scratch = [
        pltpu.VMEM((EPT,), jnp.int32),
        pltpu.VMEM((EPT,), jnp.int32),
        pltpu.VMEM((C,), jnp.int32),
        pltpu.VMEM((C,), jnp.int32),
        pltpu.VMEM((C,), jnp.int32),
        pltpu.VMEM((C, QW), jnp.float32),
        pltpu.VMEM((C, QW), jnp.float32),
        pltpu.VMEM((C, QW), jnp.float32),
        pltpu.VMEM((ZR, QW), jnp.float32),
        pltpu.VMEM_SHARED((N, QW), jnp.float32),
        pltpu.SemaphoreType.DMA,
    ]
    if with_deg:
        out_type.append(jax.ShapeDtypeStruct((N, DW), jnp.float32))
        scratch += [
            pltpu.VMEM((C, DW), jnp.float32),
            pltpu.VMEM((RPT, DW), jnp.float32),
            pltpu.VMEM_SHARED((N, DW), jnp.float32),
        ]

    def body(a_hbm, b_hbm, src_hbm, dst_hbm, acc_out, *rest):
        if with_deg:
            (degw_out, srcg_v, dst_v, ga_v, gb_v, sc_v, bufA, bufB, bufR,
             zbuf, acc_sh, sem, ones_b, z16, degw_sh) = rest
        else:
            (srcg_v, dst_v, ga_v, gb_v, sc_v, bufA, bufB, bufR,
             zbuf, acc_sh, sem) = rest
        cid = lax.axis_index("c")
        sid = lax.axis_index("s")

        zvec = jnp.zeros((16,), jnp.float32)

        def zloop(i, _):
            for r in range(QW // 16):
                zbuf[i, pl.ds(r * 16, 16)] = zvec
            return 0

        lax.fori_loop(0, ZR, zloop, 0)
        row0 = sid * RPT
        for k in range(RPT // ZR):
            pltpu.sync_copy(zbuf, acc_sh.at[pl.ds(row0 + k * ZR, ZR)])

        @pl.when(sid == NS - 1)
        def _():
            pltpu.sync_copy(zbuf.at[pl.ds(0, N - NS * RPT)],
                            acc_sh.at[pl.ds(NS * RPT, N - NS * RPT)])

        if with_deg:
            @pl.when(cid == 0)
            def _():
                def zd(i, _):
                    z16[i, :] = zvec
                    return 0
                lax.fori_loop(0, RPT, zd, 0)

                def od(i, _):
                    ones_b[i, :] = jnp.full((16,), 1.0, jnp.float32)
                    return 0
                lax.fori_loop(0, C, od, 0)
                pltpu.sync_copy(z16, degw_sh.at[pl.ds(row0, RPT)])

                @pl.when(sid == NS - 1)
                def _():
                    pltpu.sync_copy(z16.at[pl.ds(0, N - NS * RPT)],
                                    degw_sh.at[pl.ds(NS * RPT, N - NS * RPT)])

        base = sid * EPT
        pltpu.sync_copy(src_hbm.at[pl.ds(base, EPT)], srcg_v)
        pltpu.sync_copy(dst_hbm.at[pl.ds(base, EPT)], dst_v)

        def tloop(i, _):
            o = pl.ds(i * 16, 16)
            srcg_v[o] = srcg_v[o] * 2 + cid
            return 0

        lax.fori_loop(0, EPT // 16, tloop, 0)

        plsc.subcore_barrier()

        def chunk(j, _):
            cb = j * C
            for k in range(C // 16):
                o = pl.ds(k * 16, 16)
                d = dst_v[pl.ds(cb + k * 16, 16)]
                sc_v[o] = d
                ga_v[o] = d * 2 + cid
                gb_v[o] = srcg_v[pl.ds(cb + k * 16, 16)]
            pltpu.async_copy(a_hbm.at[ga_v], bufA, sem).wait()
            pltpu.async_copy(b_hbm.at[gb_v], bufB, sem).wait()

            mask16 = jnp.int32(-65536)

            def eloop(i, _):
                for u in range(2):
                    e = i * 2 + u
                    for r in range(QW // 16):
                        o = pl.ds(r * 16, 16)
                        h = jnp.maximum(bufA[e, o] + bufB[e, o], 0.0)
                        bi = plsc.bitcast(h, jnp.int32)
                        bi = (bi + 32767 + ((bi >> 16) & 1)) & mask16
                        bufR[e, o] = plsc.bitcast(bi, jnp.float32)
                return 0

            lax.fori_loop(0, C // 2, eloop, 0)
            pltpu.sync_copy(bufR, acc_sh.at[sc_v], add=True)
            if with_deg:
                @pl.when(cid == 0)
                def _():
                    pltpu.sync_copy(ones_b, degw_sh.at[sc_v], add=True)
            return 0

        lax.fori_loop(0, NCH, chunk, 0)

        plsc.subcore_barrier()

        tail0 = NS * RPT
        ntail = N - tail0
        pltpu.sync_copy(acc_sh.at[pl.ds(row0, RPT)],
                        acc_out.at[pl.ds(cid * N + row0, RPT)])

        @pl.when(sid == NS - 1)
        def _():
            pltpu.sync_copy(acc_sh.at[pl.ds(tail0, ntail)],
                            acc_out.at[pl.ds(cid * N + tail0, ntail)])

        if with_deg:
            @pl.when(cid == 0)
            def _():
                pltpu.sync_copy(degw_sh.at[pl.ds(row0, RPT)],
                                degw_out.at[pl.ds(row0, RPT)])

                @pl.when(sid == NS - 1)
                def _():
                    pltpu.sync_copy(degw_sh.at[pl.ds(tail0, ntail)],
                                    degw_out.at[pl.ds(tail0, ntail)])

    return pl.kernel(body, out_type=tuple(out_type), mesh=mesh,
                     scratch_types=scratch,
                     compiler_params=pltpu.CompilerParams(
                         use_tc_tiling_on_sc=False,
                         needs_layout_passes=False))


_sc_edge_deg = _make_sc_edge(True)


BN = 1000
G = N // BN


def _tc1_body(x_ref, w_ref, b_ref, a_ref, bm_ref):
    xb = x_ref[...]
    w = w_ref[...]
    a_ref[...] = _dg(xb, w[:, :F]) + b_ref[...]
    bm_ref[...] = _dg(xb, w[:, F:])


def _tc1(x, w1, b1):
    return pl.pallas_call(
        _tc1_body,
        grid=(G,),
        in_specs=[
            pl.BlockSpec((BN, F), lambda i: (i, 0)),
            pl.BlockSpec((H, 2 * F), lambda i: (0, 0)),
            pl.BlockSpec((1, H), lambda i: (0, 0)),
        ],
        out_specs=[
            pl.BlockSpec((BN, H), lambda i: (i, 0)),
            pl.BlockSpec((BN, H), lambda i: (i, 0)),
        ],
        out_shape=[
            jax.ShapeDtypeStruct((N, H), jnp.float32),
            jax.ShapeDtypeStruct((N, H), jnp.float32),
        ],
    )(x, w1, b1)


def _tc2_body(q0_ref, q1_ref, q2_ref, q3_ref, deg_ref, w2_ref, b2_ref,
              wn_ref, bn_ref, a_ref, bm_ref):
    accb = jnp.concatenate(
        [q0_ref[...], q1_ref[...], q2_ref[...], q3_ref[...]], axis=1)
    h = jnp.maximum(_dg_hi(accb, w2_ref[...]) + deg_ref[...] * b2_ref[...],
                    0.0)
    wn = wn_ref[...]
    a_ref[...] = _dg(h, wn[:, :H]) + bn_ref[...]
    bm_ref[...] = _dg(h, wn[:, H:])


def _tc2(accf0, accf1, deg, w2, b2, wn, bn):
    return pl.pallas_call(
        _tc2_body,
        grid=(G,),
        in_specs=[
            pl.BlockSpec((BN, QW), lambda i: (i, 0)),
            pl.BlockSpec((BN, QW), lambda i: (G + i, 0)),
            pl.BlockSpec((BN, QW), lambda i: (i, 0)),
            pl.BlockSpec((BN, QW), lambda i: (G + i, 0)),
            pl.BlockSpec((BN, 1), lambda i: (i, 0)),
            pl.BlockSpec((H, H), lambda i: (0, 0)),
            pl.BlockSpec((1, H), lambda i: (0, 0)),
            pl.BlockSpec((H, 2 * H), lambda i: (0, 0)),
            pl.BlockSpec((1, H), lambda i: (0, 0)),
        ],
        out_specs=[
            pl.BlockSpec((BN, H), lambda i: (i, 0)),
            pl.BlockSpec((BN, H), lambda i: (i, 0)),
        ],
        out_shape=[
            jax.ShapeDtypeStruct((N, H), jnp.float32),
            jax.ShapeDtypeStruct((N, H), jnp.float32),
        ],
    )(accf0, accf0, accf1, accf1, deg, w2, b2, wn, bn)


def _tc3_body(q0_ref, q1_ref, q2_ref, q3_ref, deg_ref, batch_ref,
              w2_ref, b2_ref,
              pw1_ref, pb1_ref, pw2_ref, pb2_ref,
              ow1_ref, ob1_ref, ow2_ref, ob2_ref,
              s_ref, lat_ref, pacc):
    i = pl.program_id(0)
    accb = jnp.concatenate(
        [q0_ref[...], q1_ref[...], q2_ref[...], q3_ref[...]], axis=1)
    h = jnp.maximum(_dg_hi(accb, w2_ref[...]) + deg_ref[...] * b2_ref[...],
                    0.0)
    t = jnp.maximum(_dg(h, pw1_ref[...]) + pb1_ref[...], 0.0)
    a = _dg(t, pw2_ref[...]) + pb2_ref[...]
    m = jnp.max(a, axis=-1, keepdims=True)
    ex = jnp.exp(a - m)
    sm = ex / jnp.sum(ex, axis=-1, keepdims=True)
    s_ref[...] = sm
    gi = lax.broadcasted_iota(jnp.int32, (BN, NG), 1)
    gm = (batch_ref[...] == gi).astype(jnp.float32)
    srep = jnp.concatenate([sm] * NG, axis=1)
    grep = jnp.concatenate(
        [jnp.broadcast_to(gm[:, g:g + 1], (BN, S)) for g in range(NG)],
        axis=1)
    w_assign = srep * grep

    @pl.when(i == 0)
    def _():
        pacc[...] = jnp.zeros((NG * S, H), jnp.float32)

    pacc[...] += lax.dot_general(w_assign, h, (((0,), (0,)), ((), ())),
                                 precision=lax.Precision.HIGHEST,
                                 preferred_element_type=jnp.float32)

    @pl.when(i == G - 1)
    def _():
        p = pacc[...]
        t2 = jnp.maximum(_dg(p, ow1_ref[...]) + ob1_ref[...], 0.0)
        lat_ref[...] = _dg(t2, ow2_ref[...]) + ob2_ref[...]


def _tc3(accf0, accf1, deg, batch2, w2, b2, pw1, pb1, pw2, pb2,
         ow1, ob1, ow2, ob2):
    return pl.pallas_call(
        _tc3_body,
        grid=(G,),
        in_specs=[
            pl.BlockSpec((BN, QW), lambda i: (i, 0)),
            pl.BlockSpec((BN, QW), lambda i: (G + i, 0)),
            pl.BlockSpec((BN, QW), lambda i: (i, 0)),
            pl.BlockSpec((BN, QW), lambda i: (G + i, 0)),
            pl.BlockSpec((BN, 1), lambda i: (i, 0)),
            pl.BlockSpec((BN, 1), lambda i: (i, 0)),
            pl.BlockSpec((H, H), lambda i: (0, 0)),
            pl.BlockSpec((1, H), lambda i: (0, 0)),
            pl.BlockSpec((H, H), lambda i: (0, 0)),
            pl.BlockSpec((1, H), lambda i: (0, 0)),
            pl.BlockSpec((S, H), lambda i: (0, 0)),
            pl.BlockSpec((1, S), lambda i: (0, 0)),
            pl.BlockSpec((H, H), lambda i: (0, 0)),
            pl.BlockSpec((1, H), lambda i: (0, 0)),
            pl.BlockSpec((L, H), lambda i: (0, 0)),
            pl.BlockSpec((1, L), lambda i: (0, 0)),
        ],
        out_specs=[
            pl.BlockSpec((BN, S), lambda i: (i, 0)),
            pl.BlockSpec((NG * S, L), lambda i: (0, 0)),
        ],
        out_shape=[
            jax.ShapeDtypeStruct((N, S), jnp.float32),
            jax.ShapeDtypeStruct((NG * S, L), jnp.float32),
        ],
        scratch_shapes=[pltpu.VMEM((NG * S, H), jnp.float32)],
    )(accf0, accf0, accf1, accf1, deg, batch2, w2, b2, pw1, pb1, pw2, pb2,
      ow1, ob1, ow2, ob2)


def kernel(x, edge_index, batch,
           g1w1, g1b1, g1w2, g1b2,
           g2w1, g2b1, g2w2, g2b2,
           pw1, pb1, pw2, pb2,
           ow1, ob1, ow2, ob2):
    ei = edge_index.astype(jnp.int32)
    src = ei[0]
    dst = ei[1]
    batch2 = batch.astype(jnp.int32).reshape(N, 1)

    def edge_layer(a, bm):
        accs, degws = [], []
        for k in range(2):
            av = a[:, k * 2 * QW:(k + 1) * 2 * QW].reshape(NC * N, QW)
            bv = bm[:, k * 2 * QW:(k + 1) * 2 * QW].reshape(NC * N, QW)
            acq, dw = _sc_edge_deg(av, bv, src, dst)
            accs.append(acq)
            degws.append(dw)
        return accs[0], accs[1], degws[0]

    g1w2r = g1w2.astype(jnp.bfloat16).astype(jnp.float32)
    g2w2r = g2w2.astype(jnp.bfloat16).astype(jnp.float32)

    a1, b1m = _tc1(x, g1w1, g1b1.reshape(1, H))
    accf10, accf11, degw = edge_layer(a1, b1m)
    deg = degw[:, :1]

    a2, b2m = _tc2(accf10, accf11, deg, g1w2r, g1b2.reshape(1, H),
                   g2w1, g2b1.reshape(1, H))
    accf20, accf21, _ = edge_layer(a2, b2m)

    s, lat = _tc3(accf20, accf21, deg, batch2,
                  g2w2r, g2b2.reshape(1, H),
                  pw1, pb1.reshape(1, H), pw2, pb2.reshape(1, S),
                  ow1, ob1.reshape(1, H), ow2, ob2.reshape(1, L))
    return lat.reshape(NG, S, L), s
